# baseline (speedup 1.0000x reference)
import functools

import jax
import jax.numpy as jnp
from jax import lax
from jax.experimental import pallas as pl
from jax.experimental.pallas import tpu as pltpu
from jax.experimental.pallas import tpu_sc as plsc

_N = 36864
_D = 64
_K = 1024
_CC = 0.25
_BLK = 1024
_GRID = _N // _BLK

_NC = 2
_NS = 16
_NW = _NC * _NS
_BPW = _N // _NW
_L = 16


def _tc_body(xt_ref, w_ref, qt_ref, idx_ref, loss_ref, wsq_ref, iota_ref,
             acc_ref):
    i = pl.program_id(0)
    w = w_ref[...]

    @pl.when(i == 0)
    def _init():
        wsq_ref[...] = jnp.sum(w * w, axis=1, keepdims=True)
        iota_ref[...] = lax.broadcasted_iota(
            jnp.int32, (_K, 1), 0).astype(jnp.float32)
        acc_ref[...] = jnp.zeros_like(acc_ref)

    xt = xt_ref[...]
    xsq = jnp.sum(xt * xt, axis=0, keepdims=True)
    dots2 = lax.dot_general(
        w + w, xt, (((1,), (0,)), ((), ())),
        preferred_element_type=jnp.float32)
    d = (xsq + wsq_ref[...]) - dots2
    mind = jnp.min(d, axis=0, keepdims=True)
    iota = iota_ref[...] + jnp.zeros((_K, _BLK), jnp.float32)
    idxf = jnp.min(jnp.where(d == mind, iota, float(_K)), axis=0,
                   keepdims=True)
    idx_ref[...] = idxf.astype(jnp.int32).reshape(_BLK)

    oh = jnp.where(iota == idxf, 1.0, 0.0)
    qt_ref[...] = lax.dot_general(
        w, oh, (((0,), (0,)), ((), ())),
        preferred_element_type=jnp.float32,
        precision=lax.Precision.HIGHEST)
    acc_ref[...] += jnp.sum(mind, axis=1, keepdims=True)

    @pl.when(i == _GRID - 1)
    def _fin():
        mse = acc_ref[...] * (1.0 / (_N * _D))
        loss_ref[...] = mse + _CC * mse


_tc_quantize = pl.pallas_call(
    _tc_body,
    grid=(_GRID,),
    in_specs=[
        pl.BlockSpec((_D, _BLK), lambda i: (0, i)),
        pl.BlockSpec((_K, _D), lambda i: (0, 0)),
    ],
    out_specs=[
        pl.BlockSpec((_D, _BLK), lambda i: (0, i)),
        pl.BlockSpec((_BLK,), lambda i: (i,)),
        pl.BlockSpec((1, 1), lambda i: (0, 0)),
    ],
    out_shape=[
        jax.ShapeDtypeStruct((_D, _N), jnp.float32),
        jax.ShapeDtypeStruct((_N,), jnp.int32),
        jax.ShapeDtypeStruct((1, 1), jnp.float32),
    ],
    scratch_shapes=[
        pltpu.VMEM((_K, 1), jnp.float32),
        pltpu.VMEM((_K, 1), jnp.float32),
        pltpu.VMEM((1, 1), jnp.float32),
    ],
)


@functools.cache
def _make_sc_hist():
    mesh = plsc.VectorSubcoreMesh(core_axis_name="c", subcore_axis_name="s")

    @functools.partial(
        pl.kernel,
        mesh=mesh,
        out_type=jax.ShapeDtypeStruct((_NW, _K), jnp.float32),
        scratch_types=[
            pltpu.VMEM((_BPW,), jnp.int32),
            pltpu.VMEM((_K,), jnp.float32),
        ],
        compiler_params=pltpu.CompilerParams(use_tc_tiling_on_sc=False,
                                             needs_layout_passes=False),
    )
    def _sc_hist(idx_hbm, out_hbm, idx_v, hist_v):
        cid = lax.axis_index("c")
        sid = lax.axis_index("s")
        wid = sid * _NC + cid
        pltpu.sync_copy(idx_hbm.at[pl.ds(wid * _BPW, _BPW)], idx_v)

        def _zero(j, carry):
            hist_v[pl.ds(j * _L, _L)] = jnp.zeros((_L,), jnp.float32)
            return carry
        lax.fori_loop(0, _K // _L, _zero, 0)

        ones = jnp.ones((_L,), jnp.float32)

        def _accum(j, carry):
            ii = idx_v[pl.ds(j * _L, _L)]
            plsc.addupdate_scatter(hist_v, [ii], ones)
            return carry
        lax.fori_loop(0, _BPW // _L, _accum, 0)

        pltpu.sync_copy(hist_v, out_hbm.at[wid])

    return _sc_hist


def _perp_body(cnt_ref, perp_ref):
    c = cnt_ref[...]
    p = jnp.sum(c, axis=0, keepdims=True) * (1.0 / _N)
    ent = jnp.sum(p * jnp.log(p + 1e-10), axis=1, keepdims=True)
    perp_ref[...] = jnp.exp(-ent)


_perp = pl.pallas_call(
    _perp_body,
    out_shape=jax.ShapeDtypeStruct((1, 1), jnp.float32),
)


def kernel(inputs, W):
    qt, idx, loss11 = _tc_quantize(inputs.T, W)
    cnt = _make_sc_hist()(idx)
    perp11 = _perp(cnt)
    return (qt.T, loss11[0, 0], perp11[0, 0], idx)


# speedup vs baseline: 1.9651x; 1.9651x over previous
import functools

import jax
import jax.numpy as jnp
from jax import lax
from jax.experimental import pallas as pl
from jax.experimental.pallas import tpu as pltpu
from jax.experimental.pallas import tpu_sc as plsc

_N = 36864
_D = 64
_K = 1024
_CC = 0.25
_BLK = 2048
_GRID = _N // _BLK

_NC = 2
_NS = 16
_NW = _NC * _NS
_BPW = _N // _NW
_L = 16


def _tc_body(xt_ref, w_ref, qt_ref, idx_ref, loss_ref, wsq_ref, iota_ref,
             acc_ref):
    i = pl.program_id(0)
    w = w_ref[...]

    @pl.when(i == 0)
    def _init():
        wsq_ref[...] = jnp.sum(w * w, axis=1, keepdims=True)
        iota_ref[...] = lax.broadcasted_iota(
            jnp.int32, (_K, 1), 0).astype(jnp.float32)
        acc_ref[...] = jnp.zeros_like(acc_ref)

    xt = xt_ref[...]
    xsq = jnp.sum(xt * xt, axis=0, keepdims=True)
    dots2 = lax.dot_general(
        w + w, xt, (((1,), (0,)), ((), ())),
        preferred_element_type=jnp.float32)
    d = (xsq + wsq_ref[...]) - dots2
    mind = jnp.min(d, axis=0, keepdims=True)
    iota = iota_ref[...] + jnp.zeros((_K, _BLK), jnp.float32)
    idxf = jnp.min(jnp.where(d == mind, iota, float(_K)), axis=0,
                   keepdims=True)
    idx_ref[...] = idxf.astype(jnp.int32).reshape(_BLK)

    oh = jnp.where(iota == idxf, 1.0, 0.0)
    qt_ref[...] = lax.dot_general(
        w, oh, (((0,), (0,)), ((), ())),
        preferred_element_type=jnp.float32)
    acc_ref[...] += jnp.sum(mind, axis=1, keepdims=True)

    @pl.when(i == _GRID - 1)
    def _fin():
        mse = acc_ref[...] * (1.0 / (_N * _D))
        loss_ref[...] = mse + _CC * mse


_tc_quantize = pl.pallas_call(
    _tc_body,
    grid=(_GRID,),
    in_specs=[
        pl.BlockSpec((_D, _BLK), lambda i: (0, i)),
        pl.BlockSpec((_K, _D), lambda i: (0, 0)),
    ],
    out_specs=[
        pl.BlockSpec((_D, _BLK), lambda i: (0, i)),
        pl.BlockSpec((_BLK,), lambda i: (i,)),
        pl.BlockSpec((1, 1), lambda i: (0, 0)),
    ],
    out_shape=[
        jax.ShapeDtypeStruct((_D, _N), jnp.float32),
        jax.ShapeDtypeStruct((_N,), jnp.int32),
        jax.ShapeDtypeStruct((1, 1), jnp.float32),
    ],
    scratch_shapes=[
        pltpu.VMEM((_K, 1), jnp.float32),
        pltpu.VMEM((_K, 1), jnp.float32),
        pltpu.VMEM((1, 1), jnp.float32),
    ],
)


@functools.cache
def _make_sc_hist():
    mesh = plsc.VectorSubcoreMesh(core_axis_name="c", subcore_axis_name="s")

    @functools.partial(
        pl.kernel,
        mesh=mesh,
        out_type=jax.ShapeDtypeStruct((_NW, _K), jnp.float32),
        scratch_types=[
            pltpu.VMEM((_BPW,), jnp.int32),
            pltpu.VMEM((_K,), jnp.float32),
        ],
        compiler_params=pltpu.CompilerParams(use_tc_tiling_on_sc=False,
                                             needs_layout_passes=False),
    )
    def _sc_hist(idx_hbm, out_hbm, idx_v, hist_v):
        cid = lax.axis_index("c")
        sid = lax.axis_index("s")
        wid = sid * _NC + cid
        pltpu.sync_copy(idx_hbm.at[pl.ds(wid * _BPW, _BPW)], idx_v)

        def _zero(j, carry):
            hist_v[pl.ds(j * _L, _L)] = jnp.zeros((_L,), jnp.float32)
            return carry
        lax.fori_loop(0, _K // _L, _zero, 0)

        ones = jnp.ones((_L,), jnp.float32)

        def _accum(j, carry):
            ii = idx_v[pl.ds(j * _L, _L)]
            plsc.addupdate_scatter(hist_v, [ii], ones)
            return carry
        lax.fori_loop(0, _BPW // _L, _accum, 0)

        pltpu.sync_copy(hist_v, out_hbm.at[wid])

    return _sc_hist


def _perp_body(cnt_ref, perp_ref):
    c = cnt_ref[...]
    p = jnp.sum(c, axis=0, keepdims=True) * (1.0 / _N)
    ent = jnp.sum(p * jnp.log(p + 1e-10), axis=1, keepdims=True)
    perp_ref[...] = jnp.exp(-ent)


_perp = pl.pallas_call(
    _perp_body,
    out_shape=jax.ShapeDtypeStruct((1, 1), jnp.float32),
)


def kernel(inputs, W):
    qt, idx, loss11 = _tc_quantize(inputs.T, W)
    cnt = _make_sc_hist()(idx)
    perp11 = _perp(cnt)
    return (qt.T, loss11[0, 0], perp11[0, 0], idx)


# speedup vs baseline: 1.9878x; 1.0116x over previous
import functools

import jax
import jax.numpy as jnp
from jax import lax
from jax.experimental import pallas as pl
from jax.experimental.pallas import tpu as pltpu
from jax.experimental.pallas import tpu_sc as plsc

_N = 36864
_D = 64
_K = 1024
_CC = 0.25
_BLK = 4096
_GRID = _N // _BLK

_NC = 2
_NS = 16
_NW = _NC * _NS
_BPW = _N // _NW
_L = 16


def _tc_body(xt_ref, w_ref, qt_ref, idx_ref, loss_ref, wsq_ref, iota_ref,
             acc_ref):
    i = pl.program_id(0)
    w = w_ref[...]

    @pl.when(i == 0)
    def _init():
        wsq_ref[...] = jnp.sum(w * w, axis=1, keepdims=True)
        iota_ref[...] = lax.broadcasted_iota(
            jnp.int32, (_K, 1), 0).astype(jnp.float32)
        acc_ref[...] = jnp.zeros_like(acc_ref)

    xt = xt_ref[...]
    xsq = jnp.sum(xt * xt, axis=0, keepdims=True)
    dots2 = lax.dot_general(
        w + w, xt, (((1,), (0,)), ((), ())),
        preferred_element_type=jnp.float32)
    d = (xsq + wsq_ref[...]) - dots2
    mind = jnp.min(d, axis=0, keepdims=True)
    iota = iota_ref[...] + jnp.zeros((_K, _BLK), jnp.float32)
    idxf = jnp.min(jnp.where(d == mind, iota, float(_K)), axis=0,
                   keepdims=True)
    idx_ref[...] = idxf.astype(jnp.int32).reshape(_BLK)

    oh = jnp.where(iota == idxf, 1.0, 0.0)
    qt_ref[...] = lax.dot_general(
        w, oh, (((0,), (0,)), ((), ())),
        preferred_element_type=jnp.float32)
    acc_ref[...] += jnp.sum(mind, axis=1, keepdims=True)

    @pl.when(i == _GRID - 1)
    def _fin():
        mse = acc_ref[...] * (1.0 / (_N * _D))
        loss_ref[...] = mse + _CC * mse


_tc_quantize = pl.pallas_call(
    _tc_body,
    grid=(_GRID,),
    in_specs=[
        pl.BlockSpec((_D, _BLK), lambda i: (0, i)),
        pl.BlockSpec((_K, _D), lambda i: (0, 0)),
    ],
    out_specs=[
        pl.BlockSpec((_D, _BLK), lambda i: (0, i)),
        pl.BlockSpec((_BLK,), lambda i: (i,)),
        pl.BlockSpec((1, 1), lambda i: (0, 0)),
    ],
    out_shape=[
        jax.ShapeDtypeStruct((_D, _N), jnp.float32),
        jax.ShapeDtypeStruct((_N,), jnp.int32),
        jax.ShapeDtypeStruct((1, 1), jnp.float32),
    ],
    scratch_shapes=[
        pltpu.VMEM((_K, 1), jnp.float32),
        pltpu.VMEM((_K, 1), jnp.float32),
        pltpu.VMEM((1, 1), jnp.float32),
    ],
)


@functools.cache
def _make_sc_hist():
    mesh = plsc.VectorSubcoreMesh(core_axis_name="c", subcore_axis_name="s")

    @functools.partial(
        pl.kernel,
        mesh=mesh,
        out_type=jax.ShapeDtypeStruct((_NW, _K), jnp.float32),
        scratch_types=[
            pltpu.VMEM((_BPW,), jnp.int32),
            pltpu.VMEM((_K,), jnp.float32),
        ],
        compiler_params=pltpu.CompilerParams(use_tc_tiling_on_sc=False,
                                             needs_layout_passes=False),
    )
    def _sc_hist(idx_hbm, out_hbm, idx_v, hist_v):
        cid = lax.axis_index("c")
        sid = lax.axis_index("s")
        wid = sid * _NC + cid
        pltpu.sync_copy(idx_hbm.at[pl.ds(wid * _BPW, _BPW)], idx_v)

        def _zero(j, carry):
            hist_v[pl.ds(j * _L, _L)] = jnp.zeros((_L,), jnp.float32)
            return carry
        lax.fori_loop(0, _K // _L, _zero, 0)

        ones = jnp.ones((_L,), jnp.float32)

        def _accum(j, carry):
            ii = idx_v[pl.ds(j * _L, _L)]
            plsc.addupdate_scatter(hist_v, [ii], ones)
            return carry
        lax.fori_loop(0, _BPW // _L, _accum, 0)

        pltpu.sync_copy(hist_v, out_hbm.at[wid])

    return _sc_hist


def _perp_body(cnt_ref, perp_ref):
    c = cnt_ref[...]
    p = jnp.sum(c, axis=0, keepdims=True) * (1.0 / _N)
    ent = jnp.sum(p * jnp.log(p + 1e-10), axis=1, keepdims=True)
    perp_ref[...] = jnp.exp(-ent)


_perp = pl.pallas_call(
    _perp_body,
    out_shape=jax.ShapeDtypeStruct((1, 1), jnp.float32),
)


def kernel(inputs, W):
    qt, idx, loss11 = _tc_quantize(inputs.T, W)
    cnt = _make_sc_hist()(idx)
    perp11 = _perp(cnt)
    return (qt.T, loss11[0, 0], perp11[0, 0], idx)


# speedup vs baseline: 2.0236x; 1.0180x over previous
import functools

import jax
import jax.numpy as jnp
from jax import lax
from jax.experimental import pallas as pl
from jax.experimental.pallas import tpu as pltpu
from jax.experimental.pallas import tpu_sc as plsc

_N = 36864
_D = 64
_K = 1024
_CC = 0.25
_BLK = 6144
_GRID = _N // _BLK

_NC = 2
_NS = 16
_NW = _NC * _NS
_BPW = _N // _NW
_L = 16


def _tc_body(xt_ref, w_ref, qt_ref, idx_ref, loss_ref, wsq_ref, iota_ref,
             acc_ref):
    i = pl.program_id(0)
    w = w_ref[...]

    @pl.when(i == 0)
    def _init():
        wsq_ref[...] = jnp.sum(w * w, axis=1, keepdims=True)
        iota_ref[...] = lax.broadcasted_iota(
            jnp.int32, (_K, 1), 0).astype(jnp.float32)
        acc_ref[...] = jnp.zeros_like(acc_ref)

    xt = xt_ref[...]
    xsq = jnp.sum(xt * xt, axis=0, keepdims=True)
    dots2 = lax.dot_general(
        w + w, xt, (((1,), (0,)), ((), ())),
        preferred_element_type=jnp.float32)
    d = (xsq + wsq_ref[...]) - dots2
    mind = jnp.min(d, axis=0, keepdims=True)
    iota = iota_ref[...] + jnp.zeros((_K, _BLK), jnp.float32)
    idxf = jnp.min(jnp.where(d == mind, iota, float(_K)), axis=0,
                   keepdims=True)
    idx_ref[...] = idxf.astype(jnp.int32).reshape(_BLK)

    oh = jnp.where(iota == idxf, 1.0, 0.0)
    qt_ref[...] = lax.dot_general(
        w, oh, (((0,), (0,)), ((), ())),
        preferred_element_type=jnp.float32)
    acc_ref[...] += jnp.sum(mind, axis=1, keepdims=True)

    @pl.when(i == _GRID - 1)
    def _fin():
        mse = acc_ref[...] * (1.0 / (_N * _D))
        loss_ref[...] = mse + _CC * mse


_tc_quantize = pl.pallas_call(
    _tc_body,
    grid=(_GRID,),
    in_specs=[
        pl.BlockSpec((_D, _BLK), lambda i: (0, i)),
        pl.BlockSpec((_K, _D), lambda i: (0, 0)),
    ],
    out_specs=[
        pl.BlockSpec((_D, _BLK), lambda i: (0, i)),
        pl.BlockSpec((_BLK,), lambda i: (i,)),
        pl.BlockSpec((1, 1), lambda i: (0, 0)),
    ],
    out_shape=[
        jax.ShapeDtypeStruct((_D, _N), jnp.float32),
        jax.ShapeDtypeStruct((_N,), jnp.int32),
        jax.ShapeDtypeStruct((1, 1), jnp.float32),
    ],
    scratch_shapes=[
        pltpu.VMEM((_K, 1), jnp.float32),
        pltpu.VMEM((_K, 1), jnp.float32),
        pltpu.VMEM((1, 1), jnp.float32),
    ],
    compiler_params=pltpu.CompilerParams(
        vmem_limit_bytes=100 * 1024 * 1024),
)


@functools.cache
def _make_sc_hist():
    mesh = plsc.VectorSubcoreMesh(core_axis_name="c", subcore_axis_name="s")

    @functools.partial(
        pl.kernel,
        mesh=mesh,
        out_type=jax.ShapeDtypeStruct((_NW, _K), jnp.float32),
        scratch_types=[
            pltpu.VMEM((_BPW,), jnp.int32),
            pltpu.VMEM((_K,), jnp.float32),
        ],
        compiler_params=pltpu.CompilerParams(use_tc_tiling_on_sc=False,
                                             needs_layout_passes=False),
    )
    def _sc_hist(idx_hbm, out_hbm, idx_v, hist_v):
        cid = lax.axis_index("c")
        sid = lax.axis_index("s")
        wid = sid * _NC + cid
        pltpu.sync_copy(idx_hbm.at[pl.ds(wid * _BPW, _BPW)], idx_v)

        def _zero(j, carry):
            hist_v[pl.ds(j * _L, _L)] = jnp.zeros((_L,), jnp.float32)
            return carry
        lax.fori_loop(0, _K // _L, _zero, 0)

        ones = jnp.ones((_L,), jnp.float32)

        def _accum(j, carry):
            ii = idx_v[pl.ds(j * _L, _L)]
            plsc.addupdate_scatter(hist_v, [ii], ones)
            return carry
        lax.fori_loop(0, _BPW // _L, _accum, 0)

        pltpu.sync_copy(hist_v, out_hbm.at[wid])

    return _sc_hist


def _perp_body(cnt_ref, perp_ref):
    c = cnt_ref[...]
    p = jnp.sum(c, axis=0, keepdims=True) * (1.0 / _N)
    ent = jnp.sum(p * jnp.log(p + 1e-10), axis=1, keepdims=True)
    perp_ref[...] = jnp.exp(-ent)


_perp = pl.pallas_call(
    _perp_body,
    out_shape=jax.ShapeDtypeStruct((1, 1), jnp.float32),
)


def kernel(inputs, W):
    qt, idx, loss11 = _tc_quantize(inputs.T, W)
    cnt = _make_sc_hist()(idx)
    perp11 = _perp(cnt)
    return (qt.T, loss11[0, 0], perp11[0, 0], idx)


# speedup vs baseline: 2.0350x; 1.0056x over previous
import functools

import jax
import jax.numpy as jnp
from jax import lax
from jax.experimental import pallas as pl
from jax.experimental.pallas import tpu as pltpu
from jax.experimental.pallas import tpu_sc as plsc

_N = 36864
_D = 64
_K = 1024
_CC = 0.25
_BLK = 9216
_GRID = _N // _BLK

_NC = 2
_NS = 16
_NW = _NC * _NS
_BPW = _N // _NW
_L = 16


def _tc_body(xt_ref, w_ref, qt_ref, idx_ref, loss_ref, wsq_ref, iota_ref,
             acc_ref):
    i = pl.program_id(0)
    w = w_ref[...]

    @pl.when(i == 0)
    def _init():
        wsq_ref[...] = jnp.sum(w * w, axis=1, keepdims=True)
        iota_ref[...] = lax.broadcasted_iota(
            jnp.int32, (_K, 1), 0).astype(jnp.float32)
        acc_ref[...] = jnp.zeros_like(acc_ref)

    xt = xt_ref[...]
    xsq = jnp.sum(xt * xt, axis=0, keepdims=True)
    dots2 = lax.dot_general(
        w + w, xt, (((1,), (0,)), ((), ())),
        preferred_element_type=jnp.float32)
    d = (xsq + wsq_ref[...]) - dots2
    mind = jnp.min(d, axis=0, keepdims=True)
    iota = iota_ref[...] + jnp.zeros((_K, _BLK), jnp.float32)
    idxf = jnp.min(jnp.where(d == mind, iota, float(_K)), axis=0,
                   keepdims=True)
    idx_ref[...] = idxf.astype(jnp.int32).reshape(_BLK)

    oh = jnp.where(iota == idxf, 1.0, 0.0)
    qt_ref[...] = lax.dot_general(
        w, oh, (((0,), (0,)), ((), ())),
        preferred_element_type=jnp.float32)
    acc_ref[...] += jnp.sum(mind, axis=1, keepdims=True)

    @pl.when(i == _GRID - 1)
    def _fin():
        mse = acc_ref[...] * (1.0 / (_N * _D))
        loss_ref[...] = mse + _CC * mse


_tc_quantize = pl.pallas_call(
    _tc_body,
    grid=(_GRID,),
    in_specs=[
        pl.BlockSpec((_D, _BLK), lambda i: (0, i)),
        pl.BlockSpec((_K, _D), lambda i: (0, 0)),
    ],
    out_specs=[
        pl.BlockSpec((_D, _BLK), lambda i: (0, i)),
        pl.BlockSpec((_BLK,), lambda i: (i,)),
        pl.BlockSpec((1, 1), lambda i: (0, 0)),
    ],
    out_shape=[
        jax.ShapeDtypeStruct((_D, _N), jnp.float32),
        jax.ShapeDtypeStruct((_N,), jnp.int32),
        jax.ShapeDtypeStruct((1, 1), jnp.float32),
    ],
    scratch_shapes=[
        pltpu.VMEM((_K, 1), jnp.float32),
        pltpu.VMEM((_K, 1), jnp.float32),
        pltpu.VMEM((1, 1), jnp.float32),
    ],
    compiler_params=pltpu.CompilerParams(
        vmem_limit_bytes=120 * 1024 * 1024),
)


@functools.cache
def _make_sc_hist():
    mesh = plsc.VectorSubcoreMesh(core_axis_name="c", subcore_axis_name="s")

    @functools.partial(
        pl.kernel,
        mesh=mesh,
        out_type=jax.ShapeDtypeStruct((_NW, _K), jnp.float32),
        scratch_types=[
            pltpu.VMEM((_BPW,), jnp.int32),
            pltpu.VMEM((_K,), jnp.float32),
        ],
        compiler_params=pltpu.CompilerParams(use_tc_tiling_on_sc=False,
                                             needs_layout_passes=False),
    )
    def _sc_hist(idx_hbm, out_hbm, idx_v, hist_v):
        cid = lax.axis_index("c")
        sid = lax.axis_index("s")
        wid = sid * _NC + cid
        pltpu.sync_copy(idx_hbm.at[pl.ds(wid * _BPW, _BPW)], idx_v)

        def _zero(j, carry):
            hist_v[pl.ds(j * _L, _L)] = jnp.zeros((_L,), jnp.float32)
            return carry
        lax.fori_loop(0, _K // _L, _zero, 0)

        ones = jnp.ones((_L,), jnp.float32)

        def _accum(j, carry):
            ii = idx_v[pl.ds(j * _L, _L)]
            plsc.addupdate_scatter(hist_v, [ii], ones)
            return carry
        lax.fori_loop(0, _BPW // _L, _accum, 0)

        pltpu.sync_copy(hist_v, out_hbm.at[wid])

    return _sc_hist


def _perp_body(cnt_ref, perp_ref):
    c = cnt_ref[...]
    p = jnp.sum(c, axis=0, keepdims=True) * (1.0 / _N)
    ent = jnp.sum(p * jnp.log(p + 1e-10), axis=1, keepdims=True)
    perp_ref[...] = jnp.exp(-ent)


_perp = pl.pallas_call(
    _perp_body,
    out_shape=jax.ShapeDtypeStruct((1, 1), jnp.float32),
)


def kernel(inputs, W):
    qt, idx, loss11 = _tc_quantize(inputs.T, W)
    cnt = _make_sc_hist()(idx)
    perp11 = _perp(cnt)
    return (qt.T, loss11[0, 0], perp11[0, 0], idx)
